# baseline (speedup 1.0000x reference)
import functools

import jax
import jax.numpy as jnp
from jax import lax
from jax.experimental import pallas as pl
from jax.experimental.pallas import tpu as pltpu
from jax.experimental.pallas import tpu_sc as plsc

_T = 0.0025
_M = 0.6
_K = 16
_N = 256
_L = 16
_NC = 2
_NS = 16
_NW = _NC * _NS
_RPW = _N // _NW
_NCHUNK = _N // _L


def _tc_dist_body(yi_ref, yit_ref, dii_ref, dt_ref):
    yi = yi_ref[...]
    yit = yit_ref[...]
    yin = yi * lax.rsqrt(jnp.sum(yi * yi, axis=1, keepdims=True) + 1e-12)
    yitn = yit * lax.rsqrt(jnp.sum(yit * yit, axis=1, keepdims=True) + 1e-12)
    g = lax.dot_general(yin, yin, (((1,), (1,)), ((), ())),
                        preferred_element_type=jnp.float32)
    c = lax.dot_general(yitn, yin, (((1,), (1,)), ((), ())),
                        preferred_element_type=jnp.float32)
    dii_ref[...] = 0.5 * jnp.sqrt(jnp.maximum(2.0 - 2.0 * g, 0.0) + 1e-12)
    dt_ref[...] = 0.5 * jnp.sqrt(jnp.maximum(2.0 - 2.0 * c, 0.0) + 1e-12)


def _sc_body(dii_hbm, dt_hbm, out_hbm, dii_v, dt_v, sk_v, sd_v, acc_v):
    cid = lax.axis_index("c")
    sid = lax.axis_index("s")
    wid = sid * _NC + cid
    base = wid * _RPW
    pltpu.sync_copy(dii_hbm.at[pl.ds(base, _RPW)], dii_v)
    pltpu.sync_copy(dt_hbm.at[pl.ds(base, _RPW)], dt_v)
    lane = lax.iota(jnp.int32, _L)
    total = jnp.zeros((_L,), jnp.float32)
    for r in range(_RPW):
        for c in range(_NCHUNK):
            ks, vs = plsc.sort_key_val(dii_v[r, pl.ds(c * _L, _L)],
                                       dt_v[r, pl.ds(c * _L, _L)])
            sk_v[c, :] = ks
            sd_v[c, :] = vs
        ptr = jnp.zeros((_L,), jnp.int32)
        d1 = jnp.zeros((_L,), jnp.float32)
        for t in range(_K + 1):
            heads = plsc.load_gather(sk_v, [lane, ptr])
            m_b = jnp.broadcast_to(jnp.min(heads), (_L,))
            cstar = plsc.all_reduce_ffs(heads == m_b)
            if t >= 1:
                heads_d = plsc.load_gather(sd_v, [lane, ptr])
                dt_s = jnp.sum(jnp.where(lane == cstar, heads_d, 0.0))
                dt_b = jnp.broadcast_to(dt_s, (_L,))
                diff = m_b - dt_b
                total = total + (diff * diff - _T)
                if t == 1:
                    d1 = m_b
            if t < _K:
                ptr = jnp.where(lane == cstar, ptr + 1, ptr)
        dtt = plsc.load_gather(
            dt_v, [jnp.full((_L,), r, jnp.int32),
                   jnp.broadcast_to(base + r, (_L,))])
        total = total + jnp.maximum(dtt + _M - d1, 0.0)
    acc_v[...] = jnp.where(lane == 0, total, 0.0)
    pltpu.sync_copy(acc_v, out_hbm.at[wid])


_sc_knn = pl.kernel(
    _sc_body,
    out_type=jax.ShapeDtypeStruct((_NW, _L), jnp.float32),
    mesh=plsc.VectorSubcoreMesh(core_axis_name="c", subcore_axis_name="s",
                                num_cores=_NC, num_subcores=_NS),
    scratch_types=[
        pltpu.VMEM((_RPW, _N), jnp.float32),
        pltpu.VMEM((_RPW, _N), jnp.float32),
        pltpu.VMEM((_NCHUNK, _L), jnp.float32),
        pltpu.VMEM((_NCHUNK, _L), jnp.float32),
        pltpu.VMEM((_L,), jnp.float32),
    ],
    compiler_params=pltpu.CompilerParams(needs_layout_passes=False),
)


@jax.jit
def kernel(yi, yi_t):
    dii, dt = pl.pallas_call(
        _tc_dist_body,
        out_shape=[jax.ShapeDtypeStruct((_N, _N), jnp.float32),
                   jax.ShapeDtypeStruct((_N, _N), jnp.float32)],
    )(yi, yi_t)
    parts = _sc_knn(dii, dt)
    return jnp.sum(parts)


# speedup vs baseline: 1.2224x; 1.2224x over previous
import functools

import jax
import jax.numpy as jnp
from jax import lax
from jax.experimental import pallas as pl
from jax.experimental.pallas import tpu as pltpu
from jax.experimental.pallas import tpu_sc as plsc

_T = 0.0025
_M = 0.6
_K = 16
_N = 256
_L = 16
_NC = 2
_NS = 16
_NW = _NC * _NS
_RPW = _N // _NW
_NCHUNK = _N // _L


def _tc_dist_body(yi_ref, yit_ref, dii_ref, dt_ref):
    yi = yi_ref[...]
    yit = yit_ref[...]
    yin = yi * lax.rsqrt(jnp.sum(yi * yi, axis=1, keepdims=True) + 1e-12)
    yitn = yit * lax.rsqrt(jnp.sum(yit * yit, axis=1, keepdims=True) + 1e-12)
    g = lax.dot_general(yin, yin, (((1,), (1,)), ((), ())),
                        preferred_element_type=jnp.float32)
    c = lax.dot_general(yitn, yin, (((1,), (1,)), ((), ())),
                        preferred_element_type=jnp.float32)
    dii_ref[...] = 0.5 * jnp.sqrt(jnp.maximum(2.0 - 2.0 * g, 0.0) + 1e-12)
    dt_ref[...] = 0.5 * jnp.sqrt(jnp.maximum(2.0 - 2.0 * c, 0.0) + 1e-12)


def _sc_body(dii_hbm, dt_hbm, out_hbm, dii_v, dt_v, acc_v):
    cid = lax.axis_index("c")
    sid = lax.axis_index("s")
    wid = sid * _NC + cid
    base = wid * _RPW
    pltpu.sync_copy(dii_hbm.at[pl.ds(base, _RPW)], dii_v)
    pltpu.sync_copy(dt_hbm.at[pl.ds(base, _RPW)], dt_v)
    lane = lax.iota(jnp.int32, _L)
    total = jnp.zeros((_L,), jnp.float32)
    for r in range(_RPW):
        iv = jnp.broadcast_to(base + r, (_L,))
        bk = bv = None
        for c in range(_NCHUNK):
            key = dii_v[r, pl.ds(c * _L, _L)]
            val = dt_v[r, pl.ds(c * _L, _L)]
            key = jnp.where(lane + (c * _L) == iv, 2.0, key)
            if c == 0:
                bk, bv = plsc.sort_key_val(key, val)
            else:
                ck, cv = plsc.sort_key_val(key, val, descending=True)
                keep = bk <= ck
                lk = jnp.where(keep, bk, ck)
                lv = jnp.where(keep, bv, cv)
                bk, bv = plsc.sort_key_val(lk, lv)
        diff = bk - bv
        total = total + (diff * diff - _T)
        dtt = plsc.load_gather(
            dt_v, [jnp.full((_L,), r, jnp.int32), iv])
        total = total + jnp.where(lane == 0,
                                  jnp.maximum(dtt + _M - bk, 0.0), 0.0)
    acc_v[...] = total
    pltpu.sync_copy(acc_v, out_hbm.at[wid])


_sc_knn = pl.kernel(
    _sc_body,
    out_type=jax.ShapeDtypeStruct((_NW, _L), jnp.float32),
    mesh=plsc.VectorSubcoreMesh(core_axis_name="c", subcore_axis_name="s",
                                num_cores=_NC, num_subcores=_NS),
    scratch_types=[
        pltpu.VMEM((_RPW, _N), jnp.float32),
        pltpu.VMEM((_RPW, _N), jnp.float32),
        pltpu.VMEM((_L,), jnp.float32),
    ],
    compiler_params=pltpu.CompilerParams(needs_layout_passes=False),
)


@jax.jit
def kernel(yi, yi_t):
    dii, dt = pl.pallas_call(
        _tc_dist_body,
        out_shape=[jax.ShapeDtypeStruct((_N, _N), jnp.float32),
                   jax.ShapeDtypeStruct((_N, _N), jnp.float32)],
    )(yi, yi_t)
    parts = _sc_knn(dii, dt)
    return jnp.sum(parts)
